# Initial kernel scaffold; baseline (speedup 1.0000x reference)
import jax
import jax.numpy as jnp
from jax import lax
from jax.experimental import pallas as pl
from jax.experimental.pallas import tpu as pltpu
from jax.experimental.pallas import tpu_sc as plsc

VOCAB = 1000000
HIDDEN = 32
NUM_FIELD = 26
BATCH = 16384

N_TOT = BATCH * NUM_FIELD
NW = 32
PER_W = N_TOT // NW
CHUNK = 416
ROWS_PER_FIELD = CHUNK // NUM_FIELD
NCHUNK = PER_W // CHUNK
SUB = 104
NSUB = CHUNK // SUB

_LANES = 16


def _sc_body(x_hbm, mask_hbm, w_hbm, cb_hbm, out_hbm,
             xv, cbv, wbuf, mbuf, obuf, gsem, osem):
  wid = lax.axis_index("c") * 16 + lax.axis_index("s")
  base = wid * PER_W

  pltpu.sync_copy(x_hbm.at[pl.ds(base, PER_W)], xv)
  pltpu.sync_copy(cb_hbm, cbv)

  lane = lax.iota(jnp.int32, _LANES)
  c0 = lane // 4
  c4 = c0 + 4
  bytemask = jnp.full((_LANES,), 0xFF, jnp.int32) << ((lane % 4) * 8)
  zero = jnp.zeros((_LANES,), jnp.int32)

  def chunk_body(c, _):
    off = base + c * CHUNK
    copies = []
    for s in range(NSUB):
      idx = xv.at[pl.ds(c * CHUNK + s * SUB, SUB)]
      copies.append(pltpu.async_copy(
          w_hbm.at[idx], wbuf.at[pl.ds(s * SUB, SUB)], gsem))
      copies.append(pltpu.async_copy(
          mask_hbm.at[idx], mbuf.at[pl.ds(s * SUB, SUB)], gsem))
    for cp in copies:
      cp.wait()

    for j in range(NUM_FIELD):
      cb0 = cbv[j, pl.ds(0, _LANES)]
      cb1 = cbv[j, pl.ds(_LANES, _LANES)]

      def row_body(i, _, j=j, cb0=cb0, cb1=cb1):
        r = j + NUM_FIELD * i
        r16 = jnp.full((_LANES,), r, jnp.int32)
        m0 = plsc.load_gather(mbuf, [r16, c0])
        m1 = plsc.load_gather(mbuf, [r16, c4])
        s0 = (m0 & bytemask) != zero
        s1 = (m1 & bytemask) != zero
        w0 = wbuf[r, pl.ds(0, _LANES)]
        w1 = wbuf[r, pl.ds(_LANES, _LANES)]
        obuf[r, pl.ds(0, _LANES)] = jnp.where(s0, cb0, w0)
        obuf[r, pl.ds(_LANES, _LANES)] = jnp.where(s1, cb1, w1)
        return 0

      lax.fori_loop(0, ROWS_PER_FIELD, row_body, 0)

    pltpu.async_copy(obuf, out_hbm.at[pl.ds(off, CHUNK)], osem).wait()
    return 0

  lax.fori_loop(0, NCHUNK, chunk_body, 0)


@jax.jit
def kernel(x, codebook_mask, weight, codebook):
  x_flat = x.reshape(N_TOT).astype(jnp.int32)
  mask_i32 = lax.bitcast_convert_type(
      codebook_mask.astype(jnp.uint8).reshape(VOCAB, HIDDEN // 4, 4),
      jnp.int32)

  mesh = plsc.VectorSubcoreMesh(core_axis_name="c", subcore_axis_name="s")
  out = pl.kernel(
      _sc_body,
      out_type=jax.ShapeDtypeStruct((N_TOT, HIDDEN), jnp.float32),
      mesh=mesh,
      scratch_types=[
          pltpu.VMEM((PER_W,), jnp.int32),
          pltpu.VMEM((NUM_FIELD, HIDDEN), jnp.float32),
          pltpu.VMEM((CHUNK, HIDDEN), jnp.float32),
          pltpu.VMEM((CHUNK, HIDDEN // 4), jnp.int32),
          pltpu.VMEM((CHUNK, HIDDEN), jnp.float32),
          pltpu.SemaphoreType.DMA,
          pltpu.SemaphoreType.DMA,
      ],
  )(x_flat, mask_i32, weight, codebook)
  return out.reshape(BATCH, NUM_FIELD, HIDDEN)


# speedup vs baseline: 4.8984x; 4.8984x over previous
import jax
import jax.numpy as jnp
from jax import lax
from jax.experimental import pallas as pl
from jax.experimental.pallas import tpu as pltpu
from jax.experimental.pallas import tpu_sc as plsc

VOCAB = 1000000
HIDDEN = 32
NUM_FIELD = 26
BATCH = 16384

N_TOT = BATCH * NUM_FIELD
NW = 32
PER_W = N_TOT // NW
CHUNK = 416
ROWS_PER_FIELD = CHUNK // NUM_FIELD
NCHUNK = PER_W // CHUNK
SUB = 104
NSUB = CHUNK // SUB

_LANES = 16


def _sc_body(x_hbm, mask_hbm, w_hbm, cb_hbm, out_hbm,
             xv, cbv, wbuf, mbuf, obuf, gsem, osem):
  wid = lax.axis_index("c") * 16 + lax.axis_index("s")
  base = wid * PER_W

  pltpu.sync_copy(x_hbm.at[pl.ds(base, PER_W)], xv)
  pltpu.sync_copy(cb_hbm, cbv)

  lane = lax.iota(jnp.int32, _LANES)
  c0 = lane // 4
  c4 = c0 + 4
  bytemask = jnp.full((_LANES,), 0xFF, jnp.int32) << ((lane % 4) * 8)
  zero = jnp.zeros((_LANES,), jnp.int32)

  def chunk_body(c, _):
    off = base + c * CHUNK
    copies = []
    for s in range(NSUB):
      idx = xv.at[pl.ds(c * CHUNK + s * SUB, SUB)]
      copies.append(pltpu.async_copy(
          w_hbm.at[idx], wbuf.at[pl.ds(s * SUB, SUB)], gsem))
      copies.append(pltpu.async_copy(
          mask_hbm.at[idx], mbuf.at[pl.ds(s * SUB, SUB)], gsem))
    for cp in copies:
      cp.wait()

    for j in range(NUM_FIELD):
      cb0 = cbv[j, pl.ds(0, _LANES)]
      cb1 = cbv[j, pl.ds(_LANES, _LANES)]

      def row_body(i, _, j=j, cb0=cb0, cb1=cb1):
        r = j + NUM_FIELD * i
        r16 = jnp.full((_LANES,), r, jnp.int32)
        m0 = plsc.load_gather(mbuf, [r16, c0])
        m1 = plsc.load_gather(mbuf, [r16, c4])
        s0 = (m0 & bytemask) != zero
        s1 = (m1 & bytemask) != zero
        w0 = wbuf[r, pl.ds(0, _LANES)]
        w1 = wbuf[r, pl.ds(_LANES, _LANES)]
        obuf[r, pl.ds(0, _LANES)] = jnp.where(s0, cb0, w0)
        obuf[r, pl.ds(_LANES, _LANES)] = jnp.where(s1, cb1, w1)
        return 0

      lax.fori_loop(0, ROWS_PER_FIELD, row_body, 0)

    pltpu.async_copy(obuf, out_hbm.at[pl.ds(off, CHUNK)], osem).wait()
    return 0

  lax.fori_loop(0, NCHUNK, chunk_body, 0)


@jax.jit
def kernel(x, codebook_mask, weight, codebook):
  x_flat = x.reshape(N_TOT).astype(jnp.int32)
  mask_i32 = lax.bitcast_convert_type(
      codebook_mask.astype(jnp.uint8).reshape(VOCAB, HIDDEN // 4, 4),
      jnp.int32)

  mesh = plsc.VectorSubcoreMesh(core_axis_name="c", subcore_axis_name="s")
  out = pl.kernel(
      _sc_body,
      out_type=jax.ShapeDtypeStruct((N_TOT, HIDDEN), jnp.float32),
      mesh=mesh,
      compiler_params=pltpu.CompilerParams(
          use_tc_tiling_on_sc=False, needs_layout_passes=False),
      scratch_types=[
          pltpu.VMEM((PER_W,), jnp.int32),
          pltpu.VMEM((NUM_FIELD, HIDDEN), jnp.float32),
          pltpu.VMEM((CHUNK, HIDDEN), jnp.float32),
          pltpu.VMEM((CHUNK, HIDDEN // 4), jnp.int32),
          pltpu.VMEM((CHUNK, HIDDEN), jnp.float32),
          pltpu.SemaphoreType.DMA,
          pltpu.SemaphoreType.DMA,
      ],
  )(x_flat, mask_i32, weight, codebook)
  return out.reshape(BATCH, NUM_FIELD, HIDDEN)


# speedup vs baseline: 6.8005x; 1.3883x over previous
import jax
import jax.numpy as jnp
from jax import lax
from jax.experimental import pallas as pl
from jax.experimental.pallas import tpu as pltpu
from jax.experimental.pallas import tpu_sc as plsc

VOCAB = 1000000
HIDDEN = 32
NUM_FIELD = 26
BATCH = 16384

N_TOT = BATCH * NUM_FIELD
NW = 32
PER_W = N_TOT // NW
CHUNK = 416
ROWS_PER_FIELD = CHUNK // NUM_FIELD
NCHUNK = PER_W // CHUNK
SUB = 104
NSUB = CHUNK // SUB

_LANES = 16


def _sc_body(x_hbm, mask_hbm, w_hbm, cb_hbm, out_hbm,
             xv, cbv, wbuf, mbuf, obuf, gsem, osem):
  wid = lax.axis_index("c") * 16 + lax.axis_index("s")
  base = wid * PER_W

  pltpu.sync_copy(x_hbm.at[pl.ds(base, PER_W)], xv)
  pltpu.sync_copy(cb_hbm, cbv)

  fzero = jnp.zeros((_LANES,), jnp.float32)

  def chunk_body(c, _):
    off = base + c * CHUNK
    copies = []
    for s in range(NSUB):
      idx = xv.at[pl.ds(c * CHUNK + s * SUB, SUB)]
      copies.append(pltpu.async_copy(
          w_hbm.at[idx], wbuf.at[pl.ds(s * SUB, SUB)], gsem))
      copies.append(pltpu.async_copy(
          mask_hbm.at[idx], mbuf.at[pl.ds(s * SUB, SUB)], gsem))
    for cp in copies:
      cp.wait()

    for j in range(NUM_FIELD):
      cb0 = cbv[j, pl.ds(0, _LANES)]
      cb1 = cbv[j, pl.ds(_LANES, _LANES)]

      def row_body(i, _, cb0=cb0, cb1=cb1, j=j):
        r = j + NUM_FIELD * i
        s0 = mbuf[r, pl.ds(0, _LANES)] != fzero
        s1 = mbuf[r, pl.ds(_LANES, _LANES)] != fzero
        w0 = wbuf[r, pl.ds(0, _LANES)]
        w1 = wbuf[r, pl.ds(_LANES, _LANES)]
        obuf[r, pl.ds(0, _LANES)] = jnp.where(s0, cb0, w0)
        obuf[r, pl.ds(_LANES, _LANES)] = jnp.where(s1, cb1, w1)
        return 0

      lax.fori_loop(0, ROWS_PER_FIELD, row_body, 0)

    pltpu.async_copy(obuf, out_hbm.at[pl.ds(off, CHUNK)], osem).wait()
    return 0

  lax.fori_loop(0, NCHUNK, chunk_body, 0)


@jax.jit
def kernel(x, codebook_mask, weight, codebook):
  x_flat = x.reshape(N_TOT).astype(jnp.int32)
  mask_f = codebook_mask.astype(jnp.float32)

  mesh = plsc.VectorSubcoreMesh(core_axis_name="c", subcore_axis_name="s")
  out = pl.kernel(
      _sc_body,
      out_type=jax.ShapeDtypeStruct((N_TOT, HIDDEN), jnp.float32),
      mesh=mesh,
      compiler_params=pltpu.CompilerParams(
          use_tc_tiling_on_sc=False, needs_layout_passes=False),
      scratch_types=[
          pltpu.VMEM((PER_W,), jnp.int32),
          pltpu.VMEM((NUM_FIELD, HIDDEN), jnp.float32),
          pltpu.VMEM((CHUNK, HIDDEN), jnp.float32),
          pltpu.VMEM((CHUNK, HIDDEN), jnp.float32),
          pltpu.VMEM((CHUNK, HIDDEN), jnp.float32),
          pltpu.SemaphoreType.DMA,
          pltpu.SemaphoreType.DMA,
      ],
  )(x_flat, mask_f, weight, codebook)
  return out.reshape(BATCH, NUM_FIELD, HIDDEN)
